# baseline (speedup 1.0000x reference)
import functools

import jax
import jax.numpy as jnp
from jax import lax
from jax.experimental import pallas as pl
from jax.experimental.pallas import tpu as pltpu
from jax.experimental.pallas import tpu_sc as plsc

N = 10000
DEG = 32
D = 128
E = N * DEG

NC = 2
NS = 16
NW = NC * NS
PER_W = E // NW
CH = 80
N_CHUNKS = PER_W // CH


def _linear_body(x_ref, w_ref, b_ref, ei_ref, o_ref, src_ref):
    acc = lax.dot_general(
        x_ref[...], w_ref[...],
        (((1,), (1,)), ((), ())),
        preferred_element_type=jnp.float32,
    )
    o_ref[...] = acc + b_ref[...]
    @pl.when(pl.program_id(0) == 0)
    def _():
        src_ref[...] = ei_ref[0, :]


def _linear(x, W, b, edge_index):
    BLK = 10000
    return pl.pallas_call(
        _linear_body,
        grid=(N // BLK,),
        in_specs=[
            pl.BlockSpec((BLK, D), lambda i: (i, 0)),
            pl.BlockSpec((D, D), lambda i: (0, 0)),
            pl.BlockSpec((1, D), lambda i: (0, 0)),
            pl.BlockSpec((2, E), lambda i: (0, 0)),
        ],
        out_specs=[
            pl.BlockSpec((BLK, D), lambda i: (i, 0)),
            pl.BlockSpec((E,), lambda i: (0,)),
        ],
        out_shape=[
            jax.ShapeDtypeStruct((N, D), jnp.float32),
            jax.ShapeDtypeStruct((E,), jnp.int32),
        ],
    )(x, W, b.reshape(1, D), edge_index)


NB = 4


@functools.cache
def _make_sc_gather():
    STEPS = (N_CHUNKS + NB - 1) // NB

    @functools.partial(
        pl.kernel,
        out_type=jax.ShapeDtypeStruct((E, D), jnp.float32),
        mesh=plsc.VectorSubcoreMesh(
            core_axis_name="c", subcore_axis_name="s",
            num_cores=NC, num_subcores=NS,
        ),
        scratch_types=[
            pltpu.VMEM((PER_W,), jnp.int32),
            [pltpu.VMEM((CH, D), jnp.float32) for _ in range(NB)],
            pltpu.VMEM_SHARED((N, D), jnp.float32),
            [pltpu.SemaphoreType.DMA for _ in range(NB)],
            [pltpu.SemaphoreType.DMA for _ in range(NB)],
        ],
    )
    def _sc_gather(y_hbm, src_hbm, out_hbm, idx_v, rows, y_sh, sg, ss):
        sub = lax.axis_index("s")
        wid = sub * NC + lax.axis_index("c")
        base = wid * PER_W
        pltpu.sync_copy(src_hbm.at[pl.ds(base, PER_W)], idx_v)
        @pl.when(sub < NS - 1)
        def _():
            pltpu.sync_copy(y_hbm.at[pl.ds(sub * 624, 624)],
                            y_sh.at[pl.ds(sub * 624, 624)])

        @pl.when(sub == NS - 1)
        def _():
            pltpu.sync_copy(y_hbm.at[pl.ds((NS - 1) * 624, N - (NS - 1) * 624)],
                            y_sh.at[pl.ds((NS - 1) * 624, N - (NS - 1) * 624)])

        plsc.subcore_barrier()

        def gather_start(c, b):
            pltpu.async_copy(y_sh.at[idx_v.at[pl.ds(c * CH, CH)]],
                             rows[b], sg[b])

        def gather_wait(c, b):
            pltpu.make_async_copy(y_sh.at[idx_v.at[pl.ds(c * CH, CH)]],
                                  rows[b], sg[b]).wait()

        def scatter_start(c, b):
            pltpu.async_copy(rows[b], out_hbm.at[pl.ds(base + c * CH, CH)],
                             ss[b])

        def scatter_wait(c, b):
            pltpu.make_async_copy(rows[b],
                                  out_hbm.at[pl.ds(base + c * CH, CH)],
                                  ss[b]).wait()

        def body(j, carry):
            for b in range(NB):
                c = NB * j + b

                @pl.when(jnp.logical_and(c >= NB, c < N_CHUNKS))
                def _():
                    scatter_wait(c - NB, b)

                @pl.when(c < N_CHUNKS)
                def _():
                    gather_start(c, b)

                cm = c - (NB - 1)
                bm = (b + 1) % NB

                @pl.when(jnp.logical_and(cm >= 0, cm < N_CHUNKS))
                def _():
                    gather_wait(cm, bm)
                    scatter_start(cm, bm)
            return carry

        lax.fori_loop(0, STEPS, body, 0)
        for cc in range(N_CHUNKS - NB, N_CHUNKS):
            scatter_wait(cc, cc % NB)

    return _sc_gather


def kernel(x, edge_index, W, b):
    y, src = _linear(x, W, b, edge_index)
    out = _make_sc_gather()(y, src)
    return out.reshape(N, DEG, D)


# speedup vs baseline: 1.0060x; 1.0060x over previous
import functools

import jax
import jax.numpy as jnp
from jax import lax
from jax.experimental import pallas as pl
from jax.experimental.pallas import tpu as pltpu
from jax.experimental.pallas import tpu_sc as plsc

N = 10000
DEG = 32
D = 128
E = N * DEG

NC = 2
NS = 16
NW = NC * NS
PER_W = E // NW
CH = 80
N_CHUNKS = PER_W // CH


def _linear_body(x_ref, w_ref, b_ref, ei_ref, o_ref, src_ref):
    acc = lax.dot_general(
        x_ref[...], w_ref[...],
        (((1,), (1,)), ((), ())),
        preferred_element_type=jnp.float32,
    )
    o_ref[...] = acc + b_ref[...]
    @pl.when(pl.program_id(0) == 0)
    def _():
        src_ref[...] = ei_ref[0, :]


def _linear(x, W, b, edge_index):
    BLK = 5000
    return pl.pallas_call(
        _linear_body,
        grid=(N // BLK,),
        in_specs=[
            pl.BlockSpec((BLK, D), lambda i: (i, 0)),
            pl.BlockSpec((D, D), lambda i: (0, 0)),
            pl.BlockSpec((1, D), lambda i: (0, 0)),
            pl.BlockSpec((2, E), lambda i: (0, 0)),
        ],
        out_specs=[
            pl.BlockSpec((BLK, D), lambda i: (i, 0)),
            pl.BlockSpec((E,), lambda i: (0,)),
        ],
        out_shape=[
            jax.ShapeDtypeStruct((N, D), jnp.float32),
            jax.ShapeDtypeStruct((E,), jnp.int32),
        ],
    )(x, W, b.reshape(1, D), edge_index)


NB = 4


@functools.cache
def _make_sc_gather():
    STEPS = (N_CHUNKS + NB - 1) // NB

    @functools.partial(
        pl.kernel,
        out_type=jax.ShapeDtypeStruct((E, D), jnp.float32),
        mesh=plsc.VectorSubcoreMesh(
            core_axis_name="c", subcore_axis_name="s",
            num_cores=NC, num_subcores=NS,
        ),
        scratch_types=[
            pltpu.VMEM((PER_W,), jnp.int32),
            [pltpu.VMEM((CH, D), jnp.float32) for _ in range(NB)],
            pltpu.VMEM_SHARED((N, D), jnp.float32),
            [pltpu.SemaphoreType.DMA for _ in range(NB)],
            [pltpu.SemaphoreType.DMA for _ in range(NB)],
        ],
    )
    def _sc_gather(y_hbm, src_hbm, out_hbm, idx_v, rows, y_sh, sg, ss):
        sub = lax.axis_index("s")
        wid = sub * NC + lax.axis_index("c")
        base = wid * PER_W
        pltpu.sync_copy(src_hbm.at[pl.ds(base, PER_W)], idx_v)
        @pl.when(sub < NS - 1)
        def _():
            pltpu.sync_copy(y_hbm.at[pl.ds(sub * 624, 624)],
                            y_sh.at[pl.ds(sub * 624, 624)])

        @pl.when(sub == NS - 1)
        def _():
            pltpu.sync_copy(y_hbm.at[pl.ds((NS - 1) * 624, N - (NS - 1) * 624)],
                            y_sh.at[pl.ds((NS - 1) * 624, N - (NS - 1) * 624)])

        plsc.subcore_barrier()

        def gather_start(c, b):
            pltpu.async_copy(y_sh.at[idx_v.at[pl.ds(c * CH, CH)]],
                             rows[b], sg[b])

        def gather_wait(c, b):
            pltpu.make_async_copy(y_sh.at[idx_v.at[pl.ds(c * CH, CH)]],
                                  rows[b], sg[b]).wait()

        def scatter_start(c, b):
            pltpu.async_copy(rows[b], out_hbm.at[pl.ds(base + c * CH, CH)],
                             ss[b])

        def scatter_wait(c, b):
            pltpu.make_async_copy(rows[b],
                                  out_hbm.at[pl.ds(base + c * CH, CH)],
                                  ss[b]).wait()

        def body(j, carry):
            for b in range(NB):
                c = NB * j + b

                @pl.when(jnp.logical_and(c >= NB, c < N_CHUNKS))
                def _():
                    scatter_wait(c - NB, b)

                @pl.when(c < N_CHUNKS)
                def _():
                    gather_start(c, b)

                cm = c - (NB - 1)
                bm = (b + 1) % NB

                @pl.when(jnp.logical_and(cm >= 0, cm < N_CHUNKS))
                def _():
                    gather_wait(cm, bm)
                    scatter_start(cm, bm)
            return carry

        lax.fori_loop(0, STEPS, body, 0)
        for cc in range(N_CHUNKS - NB, N_CHUNKS):
            scatter_wait(cc, cc % NB)

    return _sc_gather


def kernel(x, edge_index, W, b):
    y, src = _linear(x, W, b, edge_index)
    out = _make_sc_gather()(y, src)
    return out.reshape(N, DEG, D)
